# baseline (speedup 1.0000x reference)
import functools

import jax
import jax.numpy as jnp
from jax.experimental import pallas as pl
from jax.experimental.pallas import tpu as pltpu


def _leaky_relu(z):
    return jnp.where(z >= 0, z, 0.2 * z)


def _dense_kernel(x_ref, wc_ref, bc_ref, attd_ref, atts_ref, hh_ref, sc_ref):
    xm = (x_ref[:, 0, :] + x_ref[:, 1, :]) * 0.5
    hh = jnp.dot(xm, wc_ref[...], preferred_element_type=jnp.float32) + bc_ref[...]
    h0 = hh[:, 0:16]
    h1 = hh[:, 16:32]
    adst0 = jnp.sum(h0 * attd_ref[0:1, :], axis=1, keepdims=True)
    adst1 = jnp.sum(h1 * attd_ref[1:2, :], axis=1, keepdims=True)
    asrc0 = jnp.sum(h0 * atts_ref[0:1, :], axis=1, keepdims=True)
    asrc1 = jnp.sum(h1 * atts_ref[1:2, :], axis=1, keepdims=True)
    hh_ref[...] = hh
    sc_ref[...] = jnp.concatenate([asrc0, asrc1, adst0, adst1], axis=1)


def _self_loop_slot(nd, base):
    asrc = nd[:, base + 32:base + 34]
    adst = nd[:, base + 34:base + 36]
    w = jnp.exp(_leaky_relu(asrc + adst))
    r = nd.shape[0]
    wf = jnp.concatenate(
        [jnp.broadcast_to(w[:, 0:1], (r, 16)),
         jnp.broadcast_to(w[:, 1:2], (r, 16))], axis=1)
    num = nd[:, base:base + 32] * wf
    return jnp.concatenate([num, w, jnp.zeros((r, 30), jnp.float32)], axis=1)


def _edge_kernel(eidx_ref, node_ref, bias_ref, acc_ref, *, eb, nblocks, chunk):
    step = pl.program_id(0)
    rows = node_ref.shape[0]

    @pl.when(step == 0)
    def _init():
        def init_chunk(k, carry):
            nd = node_ref[pl.ds(k * chunk, chunk), :]
            acc_ref[pl.ds(k * chunk, chunk), :] = jnp.concatenate(
                [_self_loop_slot(nd, 0), _self_loop_slot(nd, 64)], axis=1)
            return carry

        jax.lax.fori_loop(0, rows // chunk, init_chunk, 0)

    def edge_body(e, carry):
        r = eidx_ref[0, 0, e]
        c = eidx_ref[0, 1, e]
        rr = jax.lax.div(r, 2)
        cc = jax.lax.div(c, 2)
        rslot = jax.lax.rem(r, 2)
        cslot = jax.lax.rem(c, 2)
        src = jax.lax.cond(
            rslot == 0,
            lambda: node_ref[pl.ds(rr, 1), 0:36],
            lambda: node_ref[pl.ds(rr, 1), 64:100])
        adst_c = jax.lax.cond(
            cslot == 0,
            lambda: node_ref[pl.ds(cc, 1), 34:36],
            lambda: node_ref[pl.ds(cc, 1), 98:100])
        z = src[:, 32:34] + adst_c
        w = jnp.exp(_leaky_relu(z))
        w = jnp.where(r != c, w, 0.0)
        wf = jnp.concatenate(
            [jnp.broadcast_to(w[:, 0:1], (1, 16)),
             jnp.broadcast_to(w[:, 1:2], (1, 16))], axis=1)
        val = jnp.concatenate([src[:, 0:32] * wf, w], axis=1)

        @pl.when(cslot == 0)
        def _s0():
            acc_ref[pl.ds(cc, 1), 0:34] = acc_ref[pl.ds(cc, 1), 0:34] + val

        @pl.when(cslot == 1)
        def _s1():
            acc_ref[pl.ds(cc, 1), 64:98] = acc_ref[pl.ds(cc, 1), 64:98] + val

        return carry

    jax.lax.fori_loop(0, eb, edge_body, 0)

    @pl.when(step == nblocks - 1)
    def _finalize():
        bias = bias_ref[...]

        def fin_chunk(k, carry):
            a = acc_ref[pl.ds(k * chunk, chunk), :]
            outs = []
            for base in (0, 64):
                d = a[:, base + 32:base + 34]
                df = jnp.concatenate(
                    [jnp.broadcast_to(d[:, 0:1], (chunk, 16)),
                     jnp.broadcast_to(d[:, 1:2], (chunk, 16))], axis=1)
                o = a[:, base:base + 32] / (df + 1e-16) + bias
                outs.append(jnp.concatenate(
                    [o, jnp.zeros((chunk, 32), jnp.float32)], axis=1))
            acc_ref[pl.ds(k * chunk, chunk), :] = jnp.concatenate(outs, axis=1)
            return carry

        jax.lax.fori_loop(0, rows // chunk, fin_chunk, 0)


def kernel(x, edge_index, W_lin, b_lin, W_gat, att, bias):
    n = x.shape[0]
    e = edge_index.shape[1]
    wc = W_lin @ W_gat
    bc = (b_lin @ W_gat).reshape(1, -1)
    attd = att[0, :, :16]
    atts = att[0, :, 16:]

    nt = 1024
    grid_a = (n + nt - 1) // nt
    hh, sc = pl.pallas_call(
        _dense_kernel,
        grid=(grid_a,),
        in_specs=[
            pl.BlockSpec((nt, 2, x.shape[2]), lambda i: (i, 0, 0)),
            pl.BlockSpec((wc.shape[0], wc.shape[1]), lambda i: (0, 0)),
            pl.BlockSpec((1, bc.shape[1]), lambda i: (0, 0)),
            pl.BlockSpec((2, 16), lambda i: (0, 0)),
            pl.BlockSpec((2, 16), lambda i: (0, 0)),
        ],
        out_specs=[
            pl.BlockSpec((nt, 32), lambda i: (i, 0)),
            pl.BlockSpec((nt, 4), lambda i: (i, 0)),
        ],
        out_shape=[
            jax.ShapeDtypeStruct((n, 32), jnp.float32),
            jax.ShapeDtypeStruct((n, 4), jnp.float32),
        ],
    )(x, wc, bc, attd, atts)

    rows = n // 2
    hh2 = hh.reshape(rows, 2, 32)
    sc2 = sc.reshape(rows, 2, 4)
    pad = jnp.zeros((rows, 28), jnp.float32)
    node_packed = jnp.concatenate(
        [hh2[:, 0], sc2[:, 0], pad, hh2[:, 1], sc2[:, 1], pad], axis=1)

    eb = 2000
    epad = ((e + eb - 1) // eb) * eb
    if epad != e:
        edge_index = jnp.pad(edge_index, ((0, 0), (0, epad - e)))
    nblocks = epad // eb
    eidx = edge_index.reshape(2, nblocks, eb).transpose(1, 0, 2)
    chunk = 1000 if rows % 1000 == 0 else rows

    acc = pl.pallas_call(
        functools.partial(_edge_kernel, eb=eb, nblocks=nblocks, chunk=chunk),
        grid=(nblocks,),
        in_specs=[
            pl.BlockSpec((1, 2, eb), lambda i: (i, 0, 0),
                         memory_space=pltpu.SMEM),
            pl.BlockSpec((rows, 128), lambda i: (0, 0)),
            pl.BlockSpec((1, 32), lambda i: (0, 0)),
        ],
        out_specs=pl.BlockSpec((rows, 128), lambda i: (0, 0)),
        out_shape=jax.ShapeDtypeStruct((rows, 128), jnp.float32),
    )(eidx, node_packed, bias.reshape(1, -1))

    return acc.reshape(rows, 2, 64)[:, :, 0:32].reshape(n, 32)


# speedup vs baseline: 1.0015x; 1.0015x over previous
import functools

import jax
import jax.numpy as jnp
from jax.experimental import pallas as pl
from jax.experimental.pallas import tpu as pltpu


def _leaky_relu(z):
    return jnp.where(z >= 0, z, 0.2 * z)


def _dense_kernel(x_ref, wc_ref, bc_ref, attd_ref, atts_ref, hh_ref, sc_ref):
    xm = (x_ref[:, 0, :] + x_ref[:, 1, :]) * 0.5
    hh = jnp.dot(xm, wc_ref[...], preferred_element_type=jnp.float32) + bc_ref[...]
    h0 = hh[:, 0:16]
    h1 = hh[:, 16:32]
    adst0 = jnp.sum(h0 * attd_ref[0:1, :], axis=1, keepdims=True)
    adst1 = jnp.sum(h1 * attd_ref[1:2, :], axis=1, keepdims=True)
    asrc0 = jnp.sum(h0 * atts_ref[0:1, :], axis=1, keepdims=True)
    asrc1 = jnp.sum(h1 * atts_ref[1:2, :], axis=1, keepdims=True)
    hh_ref[...] = hh
    sc_ref[...] = jnp.concatenate([asrc0, asrc1, adst0, adst1], axis=1)


def _self_loop_slot(nd, base):
    asrc = nd[:, base + 32:base + 34]
    adst = nd[:, base + 34:base + 36]
    w = jnp.exp(_leaky_relu(asrc + adst))
    r = nd.shape[0]
    wf = jnp.concatenate(
        [jnp.broadcast_to(w[:, 0:1], (r, 16)),
         jnp.broadcast_to(w[:, 1:2], (r, 16))], axis=1)
    num = nd[:, base:base + 32] * wf
    return jnp.concatenate([num, w, jnp.zeros((r, 30), jnp.float32)], axis=1)


def _edge_kernel(eidx_ref, node_ref, bias_ref, acc_ref, *, eb, nblocks, chunk):
    step = pl.program_id(0)
    rows = node_ref.shape[0]

    @pl.when(step == 0)
    def _init():
        def init_chunk(k, carry):
            nd = node_ref[pl.ds(k * chunk, chunk), :]
            acc_ref[pl.ds(k * chunk, chunk), :] = jnp.concatenate(
                [_self_loop_slot(nd, 0), _self_loop_slot(nd, 64)], axis=1)
            return carry

        jax.lax.fori_loop(0, rows // chunk, init_chunk, 0)

    def edge_body(e, carry):
        r = eidx_ref[0, 0, e]
        c = eidx_ref[0, 1, e]
        rr = jax.lax.div(r, 2)
        cc = jax.lax.div(c, 2)
        rslot = jax.lax.rem(r, 2)
        cslot = jax.lax.rem(c, 2)
        srow = node_ref[pl.ds(rr, 1), :]
        crow = node_ref[pl.ds(cc, 1), :]
        src = jnp.where(rslot == 0, srow[:, 0:36], srow[:, 64:100])
        adst_c = jnp.where(cslot == 0, crow[:, 34:36], crow[:, 98:100])
        z = src[:, 32:34] + adst_c
        w = jnp.exp(_leaky_relu(z))
        w = jnp.where(r != c, w, 0.0)
        wf = jnp.concatenate(
            [jnp.broadcast_to(w[:, 0:1], (1, 16)),
             jnp.broadcast_to(w[:, 1:2], (1, 16))], axis=1)
        val = jnp.concatenate([src[:, 0:32] * wf, w], axis=1)
        z30 = jnp.zeros((1, 30), jnp.float32)
        z64 = jnp.zeros((1, 64), jnp.float32)
        v128 = jnp.where(cslot == 0,
                         jnp.concatenate([val, z30, z64], axis=1),
                         jnp.concatenate([z64, val, z30], axis=1))
        acc_ref[pl.ds(cc, 1), :] = acc_ref[pl.ds(cc, 1), :] + v128

        return carry

    jax.lax.fori_loop(0, eb, edge_body, 0)

    @pl.when(step == nblocks - 1)
    def _finalize():
        bias = bias_ref[...]

        def fin_chunk(k, carry):
            a = acc_ref[pl.ds(k * chunk, chunk), :]
            outs = []
            for base in (0, 64):
                d = a[:, base + 32:base + 34]
                df = jnp.concatenate(
                    [jnp.broadcast_to(d[:, 0:1], (chunk, 16)),
                     jnp.broadcast_to(d[:, 1:2], (chunk, 16))], axis=1)
                o = a[:, base:base + 32] / (df + 1e-16) + bias
                outs.append(jnp.concatenate(
                    [o, jnp.zeros((chunk, 32), jnp.float32)], axis=1))
            acc_ref[pl.ds(k * chunk, chunk), :] = jnp.concatenate(outs, axis=1)
            return carry

        jax.lax.fori_loop(0, rows // chunk, fin_chunk, 0)


def kernel(x, edge_index, W_lin, b_lin, W_gat, att, bias):
    n = x.shape[0]
    e = edge_index.shape[1]
    wc = W_lin @ W_gat
    bc = (b_lin @ W_gat).reshape(1, -1)
    attd = att[0, :, :16]
    atts = att[0, :, 16:]

    nt = 1024
    grid_a = (n + nt - 1) // nt
    hh, sc = pl.pallas_call(
        _dense_kernel,
        grid=(grid_a,),
        in_specs=[
            pl.BlockSpec((nt, 2, x.shape[2]), lambda i: (i, 0, 0)),
            pl.BlockSpec((wc.shape[0], wc.shape[1]), lambda i: (0, 0)),
            pl.BlockSpec((1, bc.shape[1]), lambda i: (0, 0)),
            pl.BlockSpec((2, 16), lambda i: (0, 0)),
            pl.BlockSpec((2, 16), lambda i: (0, 0)),
        ],
        out_specs=[
            pl.BlockSpec((nt, 32), lambda i: (i, 0)),
            pl.BlockSpec((nt, 4), lambda i: (i, 0)),
        ],
        out_shape=[
            jax.ShapeDtypeStruct((n, 32), jnp.float32),
            jax.ShapeDtypeStruct((n, 4), jnp.float32),
        ],
    )(x, wc, bc, attd, atts)

    rows = n // 2
    hh2 = hh.reshape(rows, 2, 32)
    sc2 = sc.reshape(rows, 2, 4)
    pad = jnp.zeros((rows, 28), jnp.float32)
    node_packed = jnp.concatenate(
        [hh2[:, 0], sc2[:, 0], pad, hh2[:, 1], sc2[:, 1], pad], axis=1)

    eb = 2000
    epad = ((e + eb - 1) // eb) * eb
    if epad != e:
        edge_index = jnp.pad(edge_index, ((0, 0), (0, epad - e)))
    nblocks = epad // eb
    eidx = edge_index.reshape(2, nblocks, eb).transpose(1, 0, 2)
    chunk = 1000 if rows % 1000 == 0 else rows

    acc = pl.pallas_call(
        functools.partial(_edge_kernel, eb=eb, nblocks=nblocks, chunk=chunk),
        grid=(nblocks,),
        in_specs=[
            pl.BlockSpec((1, 2, eb), lambda i: (i, 0, 0),
                         memory_space=pltpu.SMEM),
            pl.BlockSpec((rows, 128), lambda i: (0, 0)),
            pl.BlockSpec((1, 32), lambda i: (0, 0)),
        ],
        out_specs=pl.BlockSpec((rows, 128), lambda i: (0, 0)),
        out_shape=jax.ShapeDtypeStruct((rows, 128), jnp.float32),
    )(eidx, node_packed, bias.reshape(1, -1))

    return acc.reshape(rows, 2, 64)[:, :, 0:32].reshape(n, 32)
